# Initial kernel scaffold; baseline (speedup 1.0000x reference)
import jax
import jax.numpy as jnp
from jax.experimental import pallas as pl


def kernel(x, Wgate_r, Wup_r, extra_scale, extra_bias, Wg, Wu, Wd):
    raise NotImplementedError("write your pallas kernel here")


# speedup vs baseline: 2.0342x; 2.0342x over previous
import functools

import jax
import jax.numpy as jnp
from jax.experimental import pallas as pl
from jax.experimental.pallas import tpu as pltpu


def _moe_body(x_ref, wgr_ref, wur_ref, sb_ref, wg_ref, wu_ref, wd_ref,
              y_ref, wfull_ref):
    e = pl.program_id(0)
    E = wfull_ref.shape[1]

    @pl.when(e == 0)
    def _router():
        xs = x_ref[...]
        g = jnp.dot(xs, wgr_ref[...].T, preferred_element_type=jnp.float32)
        u = jnp.dot(xs, wur_ref[...].T, preferred_element_type=jnp.float32)
        s = jnp.abs(u * (g * jax.nn.sigmoid(g)))
        s = jax.nn.softmax(s, axis=-1)
        scale = sb_ref[0:1, :]
        bias = sb_ref[1:2, :]
        sbias = s + bias
        iota = jax.lax.broadcasted_iota(jnp.int32, s.shape, 1)
        m1 = jnp.max(sbias, axis=1, keepdims=True)
        i1 = jnp.min(jnp.where(sbias == m1, iota, E), axis=1, keepdims=True)
        oh1 = iota == i1
        sb2 = jnp.where(oh1, -jnp.inf, sbias)
        m2 = jnp.max(sb2, axis=1, keepdims=True)
        i2 = jnp.min(jnp.where((sb2 == m2) & (~oh1), iota, E), axis=1,
                     keepdims=True)
        sel = oh1 | (iota == i2)
        wfull_ref[...] = jnp.where(sel, 1.0 + s * scale, 0.0)

    xb = x_ref[...].astype(jnp.bfloat16)
    wg = wg_ref[0].astype(jnp.bfloat16)
    wu = wu_ref[0].astype(jnp.bfloat16)
    wd = wd_ref[0].astype(jnp.bfloat16)
    dn = (((1,), (1,)), ((), ()))
    g = jax.lax.dot_general(xb, wg, dn, preferred_element_type=jnp.float32)
    u = jax.lax.dot_general(xb, wu, dn, preferred_element_type=jnp.float32)
    h = (g * jax.nn.sigmoid(g) * u).astype(jnp.bfloat16)
    o = jax.lax.dot_general(h, wd, dn, preferred_element_type=jnp.float32)

    iota = jax.lax.broadcasted_iota(jnp.int32, wfull_ref.shape, 1)
    w_col = jnp.sum(jnp.where(iota == e, wfull_ref[...], 0.0), axis=1,
                    keepdims=True)

    @pl.when(e == 0)
    def _init():
        y_ref[...] = o * w_col

    @pl.when(e > 0)
    def _acc():
        y_ref[...] += o * w_col


@jax.jit
def kernel(x, Wgate_r, Wup_r, extra_scale, extra_bias, Wg, Wu, Wd):
    T, D = x.shape
    E, INTER, _ = Wg.shape
    sb = jnp.stack([extra_scale, extra_bias])
    grid = (E,)
    return pl.pallas_call(
        _moe_body,
        grid=grid,
        in_specs=[
            pl.BlockSpec((T, D), lambda e: (0, 0)),
            pl.BlockSpec((E, D), lambda e: (0, 0)),
            pl.BlockSpec((E, D), lambda e: (0, 0)),
            pl.BlockSpec((2, E), lambda e: (0, 0)),
            pl.BlockSpec((1, INTER, D), lambda e: (e, 0, 0)),
            pl.BlockSpec((1, INTER, D), lambda e: (e, 0, 0)),
            pl.BlockSpec((1, D, INTER), lambda e: (e, 0, 0)),
        ],
        out_specs=pl.BlockSpec((T, D), lambda e: (0, 0)),
        out_shape=jax.ShapeDtypeStruct((T, D), jnp.float32),
        scratch_shapes=[pltpu.VMEM((T, E), jnp.float32)],
        compiler_params=pltpu.CompilerParams(
            dimension_semantics=("arbitrary",),
        ),
    )(x, Wgate_r, Wup_r, sb, Wg, Wu, Wd)


# speedup vs baseline: 2.0364x; 1.0011x over previous
import functools

import jax
import jax.numpy as jnp
from jax.experimental import pallas as pl
from jax.experimental.pallas import tpu as pltpu


def _moe_body(x_ref, wgr_ref, wur_ref, sb_ref, wg_ref, wu_ref, wd_ref,
              y_ref, wfull_ref, xb_ref):
    e = pl.program_id(0)
    E = wfull_ref.shape[1]

    @pl.when(e == 0)
    def _router():
        xs = x_ref[...]
        xb_ref[...] = xs.astype(jnp.bfloat16)
        g = jnp.dot(xs, wgr_ref[...].T, preferred_element_type=jnp.float32)
        u = jnp.dot(xs, wur_ref[...].T, preferred_element_type=jnp.float32)
        s = jnp.abs(u * (g * jax.nn.sigmoid(g)))
        s = jax.nn.softmax(s, axis=-1)
        scale = sb_ref[0:1, :]
        bias = sb_ref[1:2, :]
        sbias = s + bias
        iota = jax.lax.broadcasted_iota(jnp.int32, s.shape, 1)
        m1 = jnp.max(sbias, axis=1, keepdims=True)
        i1 = jnp.min(jnp.where(sbias == m1, iota, E), axis=1, keepdims=True)
        oh1 = iota == i1
        sb2 = jnp.where(oh1, -jnp.inf, sbias)
        m2 = jnp.max(sb2, axis=1, keepdims=True)
        i2 = jnp.min(jnp.where((sb2 == m2) & (~oh1), iota, E), axis=1,
                     keepdims=True)
        sel = oh1 | (iota == i2)
        wfull_ref[...] = jnp.where(sel, 1.0 + s * scale, 0.0)

    xb = xb_ref[...]
    wg = wg_ref[0].astype(jnp.bfloat16)
    wu = wu_ref[0].astype(jnp.bfloat16)
    wd = wd_ref[0].astype(jnp.bfloat16)
    dn = (((1,), (1,)), ((), ()))
    g = jax.lax.dot_general(xb, wg, dn, preferred_element_type=jnp.float32)
    u = jax.lax.dot_general(xb, wu, dn, preferred_element_type=jnp.float32)
    h = (g * jax.nn.sigmoid(g) * u).astype(jnp.bfloat16)
    o = jax.lax.dot_general(h, wd, dn, preferred_element_type=jnp.float32)

    iota = jax.lax.broadcasted_iota(jnp.int32, wfull_ref.shape, 1)
    w_col = jnp.sum(jnp.where(iota == e, wfull_ref[...], 0.0), axis=1,
                    keepdims=True)

    @pl.when(e == 0)
    def _init():
        y_ref[...] = o * w_col

    @pl.when(e > 0)
    def _acc():
        y_ref[...] += o * w_col


@jax.jit
def kernel(x, Wgate_r, Wup_r, extra_scale, extra_bias, Wg, Wu, Wd):
    T, D = x.shape
    E, INTER, _ = Wg.shape
    sb = jnp.stack([extra_scale, extra_bias])
    grid = (E,)
    return pl.pallas_call(
        _moe_body,
        grid=grid,
        in_specs=[
            pl.BlockSpec((T, D), lambda e: (0, 0)),
            pl.BlockSpec((E, D), lambda e: (0, 0)),
            pl.BlockSpec((E, D), lambda e: (0, 0)),
            pl.BlockSpec((2, E), lambda e: (0, 0)),
            pl.BlockSpec((1, INTER, D), lambda e: (e, 0, 0)),
            pl.BlockSpec((1, INTER, D), lambda e: (e, 0, 0)),
            pl.BlockSpec((1, D, INTER), lambda e: (e, 0, 0)),
        ],
        out_specs=pl.BlockSpec((T, D), lambda e: (0, 0)),
        out_shape=jax.ShapeDtypeStruct((T, D), jnp.float32),
        scratch_shapes=[pltpu.VMEM((T, E), jnp.float32),
                        pltpu.VMEM((T, D), jnp.bfloat16)],
        compiler_params=pltpu.CompilerParams(
            dimension_semantics=("arbitrary",),
        ),
    )(x, Wgate_r, Wup_r, sb, Wg, Wu, Wd)
